# Initial kernel scaffold; baseline (speedup 1.0000x reference)
import functools

import jax
import jax.numpy as jnp
import numpy as np
from jax import lax
from jax.experimental import pallas as pl
from jax.experimental.pallas import tpu as pltpu
from jax.experimental.pallas import tpu_sc as plsc

N, E, D, L = 10000, 320000, 128, 3

NC, NS = 2, 16
NW = NC * NS
CHUNK = 80
CPT = E // (NW * CHUNK)
RPT = N // NS


@functools.partial(
    pl.kernel,
    out_type=jax.ShapeDtypeStruct((NC, N, D), jnp.float32),
    mesh=plsc.VectorSubcoreMesh(core_axis_name="c", subcore_axis_name="s"),
    scratch_types=[
        pltpu.VMEM((CPT, CHUNK), jnp.int32),
        pltpu.VMEM((CPT, CHUNK), jnp.int32),
        pltpu.VMEM((CHUNK, D), jnp.float32),
        pltpu.VMEM_SHARED((N, D), jnp.float32),
        pltpu.SemaphoreType.DMA,
    ],
)
def _sc_agg(src_hbm, dst_hbm, h_hbm, zeros_hbm, out_hbm,
            sidx_v, didx_v, rows_v, agg_sh, sem):
    c = lax.axis_index("c")
    s = lax.axis_index("s")
    wid = s * NC + c

    pltpu.sync_copy(zeros_hbm, agg_sh.at[pl.ds(s * RPT, RPT)])
    pltpu.sync_copy(src_hbm.at[wid], sidx_v)
    pltpu.sync_copy(dst_hbm.at[wid], didx_v)
    plsc.subcore_barrier()

    def step(t, carry):
        pltpu.async_copy(h_hbm.at[sidx_v.at[t]], rows_v, sem).wait()
        pltpu.sync_copy(rows_v, agg_sh.at[didx_v.at[t]], add=True)
        return carry

    lax.fori_loop(0, CPT, step, 0)
    plsc.subcore_barrier()

    pltpu.sync_copy(agg_sh.at[pl.ds(s * RPT, RPT)],
                    out_hbm.at[c, pl.ds(s * RPT, RPT)])


_INV_SQRT2 = np.float32(1.0 / np.sqrt(2.0))


def _layer_body(eps_ref, h_ref, p_ref, w1_ref, b1_ref, w2_ref, b2_ref,
                lnw_ref, lnb_ref, o_ref):
    h = h_ref[...]
    z = (1.0 + eps_ref[0, 0]) * h + p_ref[0] + p_ref[1]
    a = jnp.maximum(
        jnp.dot(z, w1_ref[...], preferred_element_type=jnp.float32)
        + b1_ref[...], 0.0)
    z2 = (jnp.dot(a, w2_ref[...], preferred_element_type=jnp.float32)
          + b2_ref[...])
    mu = jnp.mean(z2, axis=-1, keepdims=True)
    var = jnp.mean((z2 - mu) ** 2, axis=-1, keepdims=True)
    zn = (z2 - mu) / jnp.sqrt(var + 1e-5) * lnw_ref[...] + lnb_ref[...]
    g = zn * 0.5 * (1.0 + lax.erf(zn * _INV_SQRT2))
    o_ref[...] = g + h


def _tc_layer(h, p, eps_i, w1t, b1, w2t, b2, lnw_i, lnb_i, block_n):
    grid = (N // block_n,)
    return pl.pallas_call(
        _layer_body,
        grid=grid,
        in_specs=[
            pl.BlockSpec(memory_space=pltpu.SMEM),
            pl.BlockSpec((block_n, D), lambda i: (i, 0)),
            pl.BlockSpec((NC, block_n, D), lambda i: (0, i, 0)),
            pl.BlockSpec((D, D), lambda i: (0, 0)),
            pl.BlockSpec((1, D), lambda i: (0, 0)),
            pl.BlockSpec((D, D), lambda i: (0, 0)),
            pl.BlockSpec((1, D), lambda i: (0, 0)),
            pl.BlockSpec((1, D), lambda i: (0, 0)),
            pl.BlockSpec((1, D), lambda i: (0, 0)),
        ],
        out_specs=pl.BlockSpec((block_n, D), lambda i: (i, 0)),
        out_shape=jax.ShapeDtypeStruct((N, D), jnp.float32),
        compiler_params=pltpu.CompilerParams(
            dimension_semantics=("arbitrary",)),
    )(eps_i, h, p, w1t, b1, w2t, b2, lnw_i, lnb_i)


def _final_body(h1_ref, h2_ref, h3_ref, lnw_ref, lnb_ref, wp_ref, bp_ref,
                o_ref):
    ssum = h1_ref[...] + h2_ref[...] + h3_ref[...]
    mu = jnp.mean(ssum, axis=-1, keepdims=True)
    var = jnp.mean((ssum - mu) ** 2, axis=-1, keepdims=True)
    zn = (ssum - mu) / jnp.sqrt(var + 1e-5) * lnw_ref[...] + lnb_ref[...]
    o_ref[...] = (jnp.dot(zn, wp_ref[...], preferred_element_type=jnp.float32)
                  + bp_ref[...])


def _tc_final(h1, h2, h3, lnw_f, lnb_f, wpt, bp, block_n):
    grid = (N // block_n,)
    row = pl.BlockSpec((block_n, D), lambda i: (i, 0))
    cst = pl.BlockSpec((1, D), lambda i: (0, 0))
    return pl.pallas_call(
        _final_body,
        grid=grid,
        in_specs=[row, row, row, cst, cst,
                  pl.BlockSpec((D, D), lambda i: (0, 0)), cst],
        out_specs=row,
        out_shape=jax.ShapeDtypeStruct((N, D), jnp.float32),
        compiler_params=pltpu.CompilerParams(
            dimension_semantics=("arbitrary",)),
    )(h1, h2, h3, lnw_f, lnb_f, wpt, bp)


def kernel(x, edge_index, W1, b1, W2, b2, eps, lnw, lnb, lnw_f, lnb_f, Wp, bp):
    ei = edge_index.astype(jnp.int32).reshape(2, NW, CPT, CHUNK)
    src3d, dst3d = ei[0], ei[1]
    zeros = jnp.zeros((RPT, D), jnp.float32)
    w1t = jnp.transpose(W1, (0, 2, 1))
    w2t = jnp.transpose(W2, (0, 2, 1))
    b1r = b1.reshape(L, 1, D)
    b2r = b2.reshape(L, 1, D)
    lnwr = lnw.reshape(L, 1, D)
    lnbr = lnb.reshape(L, 1, D)
    epsr = eps.reshape(L, 1, 1)

    block_n = 1000
    h = x
    hs = []
    for i in range(L):
        p = _sc_agg(src3d, dst3d, h, zeros)
        h = _tc_layer(h, p, epsr[i], w1t[i], b1r[i], w2t[i], b2r[i],
                      lnwr[i], lnbr[i], block_n)
        hs.append(h)
    return _tc_final(hs[0], hs[1], hs[2], lnw_f.reshape(1, D),
                     lnb_f.reshape(1, D), Wp.T, bp.reshape(1, D), block_n)


# speedup vs baseline: 6.4833x; 6.4833x over previous
import functools

import jax
import jax.numpy as jnp
import numpy as np
from jax import lax
from jax.experimental import pallas as pl
from jax.experimental.pallas import tpu as pltpu
from jax.experimental.pallas import tpu_sc as plsc

N, E, D, L = 10000, 320000, 128, 3

NC, NS = 2, 16
NW = NC * NS
CHUNK = 80
CPT = E // (NW * CHUNK)
RPT = 624
RPT_LAST = N - (NS - 1) * RPT


@functools.cache
def _make_sc_agg():
    @functools.partial(
        pl.kernel,
        out_type=jax.ShapeDtypeStruct((NC, N, D), jnp.float32),
        mesh=plsc.VectorSubcoreMesh(core_axis_name="c", subcore_axis_name="s"),
        scratch_types=[
            pltpu.VMEM((CPT, CHUNK), jnp.int32),
            pltpu.VMEM((CPT, CHUNK), jnp.int32),
            pltpu.VMEM((CHUNK, D), jnp.float32),
            pltpu.VMEM_SHARED((N, D), jnp.float32),
            pltpu.SemaphoreType.DMA,
        ],
    )
    def _sc_agg(src_hbm, dst_hbm, h_hbm, zeros_hbm, out_hbm,
                sidx_v, didx_v, rows_v, agg_sh, sem):
        c = lax.axis_index("c")
        s = lax.axis_index("s")
        wid = s * NC + c

        @pl.when(s < NS - 1)
        def _():
            pltpu.sync_copy(zeros_hbm.at[pl.ds(0, RPT)],
                            agg_sh.at[pl.ds(s * RPT, RPT)])

        @pl.when(s == NS - 1)
        def _():
            pltpu.sync_copy(zeros_hbm,
                            agg_sh.at[pl.ds((NS - 1) * RPT, RPT_LAST)])
        pltpu.sync_copy(src_hbm.at[wid], sidx_v)
        pltpu.sync_copy(dst_hbm.at[wid], didx_v)
        plsc.subcore_barrier()

        def step(t, carry):
            pltpu.async_copy(h_hbm.at[sidx_v.at[t]], rows_v, sem).wait()
            pltpu.sync_copy(rows_v, agg_sh.at[didx_v.at[t]], add=True)
            return carry

        lax.fori_loop(0, CPT, step, 0)
        plsc.subcore_barrier()

        @pl.when(s < NS - 1)
        def _():
            pltpu.sync_copy(agg_sh.at[pl.ds(s * RPT, RPT)],
                            out_hbm.at[c, pl.ds(s * RPT, RPT)])

        @pl.when(s == NS - 1)
        def _():
            pltpu.sync_copy(agg_sh.at[pl.ds((NS - 1) * RPT, RPT_LAST)],
                            out_hbm.at[c, pl.ds((NS - 1) * RPT, RPT_LAST)])

    return _sc_agg


_INV_SQRT2 = np.float32(1.0 / np.sqrt(2.0))


def _layer_body(eps_ref, h_ref, p_ref, w1_ref, b1_ref, w2_ref, b2_ref,
                lnw_ref, lnb_ref, o_ref):
    h = h_ref[...]
    z = (1.0 + eps_ref[0, 0]) * h + p_ref[0] + p_ref[1]
    a = jnp.maximum(
        jnp.dot(z, w1_ref[...], preferred_element_type=jnp.float32)
        + b1_ref[...], 0.0)
    z2 = (jnp.dot(a, w2_ref[...], preferred_element_type=jnp.float32)
          + b2_ref[...])
    mu = jnp.mean(z2, axis=-1, keepdims=True)
    var = jnp.mean((z2 - mu) ** 2, axis=-1, keepdims=True)
    zn = (z2 - mu) / jnp.sqrt(var + 1e-5) * lnw_ref[...] + lnb_ref[...]
    g = zn * 0.5 * (1.0 + lax.erf(zn * _INV_SQRT2))
    o_ref[...] = g + h


def _tc_layer(h, p, eps_i, w1t, b1, w2t, b2, lnw_i, lnb_i, block_n):
    grid = (N // block_n,)
    return pl.pallas_call(
        _layer_body,
        grid=grid,
        in_specs=[
            pl.BlockSpec(memory_space=pltpu.SMEM),
            pl.BlockSpec((block_n, D), lambda i: (i, 0)),
            pl.BlockSpec((NC, block_n, D), lambda i: (0, i, 0)),
            pl.BlockSpec((D, D), lambda i: (0, 0)),
            pl.BlockSpec((1, D), lambda i: (0, 0)),
            pl.BlockSpec((D, D), lambda i: (0, 0)),
            pl.BlockSpec((1, D), lambda i: (0, 0)),
            pl.BlockSpec((1, D), lambda i: (0, 0)),
            pl.BlockSpec((1, D), lambda i: (0, 0)),
        ],
        out_specs=pl.BlockSpec((block_n, D), lambda i: (i, 0)),
        out_shape=jax.ShapeDtypeStruct((N, D), jnp.float32),
        compiler_params=pltpu.CompilerParams(
            dimension_semantics=("arbitrary",)),
    )(eps_i, h, p, w1t, b1, w2t, b2, lnw_i, lnb_i)


def _final_body(h1_ref, h2_ref, h3_ref, lnw_ref, lnb_ref, wp_ref, bp_ref,
                o_ref):
    ssum = h1_ref[...] + h2_ref[...] + h3_ref[...]
    mu = jnp.mean(ssum, axis=-1, keepdims=True)
    var = jnp.mean((ssum - mu) ** 2, axis=-1, keepdims=True)
    zn = (ssum - mu) / jnp.sqrt(var + 1e-5) * lnw_ref[...] + lnb_ref[...]
    o_ref[...] = (jnp.dot(zn, wp_ref[...], preferred_element_type=jnp.float32)
                  + bp_ref[...])


def _tc_final(h1, h2, h3, lnw_f, lnb_f, wpt, bp, block_n):
    grid = (N // block_n,)
    row = pl.BlockSpec((block_n, D), lambda i: (i, 0))
    cst = pl.BlockSpec((1, D), lambda i: (0, 0))
    return pl.pallas_call(
        _final_body,
        grid=grid,
        in_specs=[row, row, row, cst, cst,
                  pl.BlockSpec((D, D), lambda i: (0, 0)), cst],
        out_specs=row,
        out_shape=jax.ShapeDtypeStruct((N, D), jnp.float32),
        compiler_params=pltpu.CompilerParams(
            dimension_semantics=("arbitrary",)),
    )(h1, h2, h3, lnw_f, lnb_f, wpt, bp)


def kernel(x, edge_index, W1, b1, W2, b2, eps, lnw, lnb, lnw_f, lnb_f, Wp, bp):
    ei = edge_index.astype(jnp.int32).reshape(2, NW, CPT, CHUNK)
    src3d, dst3d = ei[0], ei[1]
    zeros = jnp.zeros((RPT_LAST, D), jnp.float32)
    w1t = jnp.transpose(W1, (0, 2, 1))
    w2t = jnp.transpose(W2, (0, 2, 1))
    b1r = b1.reshape(L, 1, D)
    b2r = b2.reshape(L, 1, D)
    lnwr = lnw.reshape(L, 1, D)
    lnbr = lnb.reshape(L, 1, D)
    epsr = eps.reshape(L, 1, 1)

    block_n = 1000
    h = x
    hs = []
    for i in range(L):
        p = _make_sc_agg()(src3d, dst3d, h, zeros)
        h = _tc_layer(h, p, epsr[i], w1t[i], b1r[i], w2t[i], b2r[i],
                      lnwr[i], lnbr[i], block_n)
        hs.append(h)
    return _tc_final(hs[0], hs[1], hs[2], lnw_f.reshape(1, D),
                     lnb_f.reshape(1, D), Wp.T, bp.reshape(1, D), block_n)
